# Initial kernel scaffold; baseline (speedup 1.0000x reference)
import jax
import jax.numpy as jnp
from jax.experimental import pallas as pl


def kernel(x):
    raise NotImplementedError("write your pallas kernel here")


# speedup vs baseline: 16901.3406x; 16901.3406x over previous
import jax
import jax.numpy as jnp
import numpy as np
from jax.experimental import pallas as pl

STRICT_MAXIMA_BONUS = 10.0
EPS = 1e-07

def _compute_pert():
    def gen():
        v = jax.random.uniform(
            jax.random.fold_in(jax.random.key(0), 7), (3, 3), dtype=jnp.float32)
        return np.abs(np.asarray(v)) * EPS
    try:
        with jax.default_device(jax.local_devices(backend="cpu")[0]):
            v = gen()
    except Exception:
        v = gen()
    return [[float(z) for z in row] for row in v]


_PERT = _compute_pert()


def _pert():
    return _PERT


def _make_body(D, H, W, TH, dtype, p):
    neg = float('-inf')

    def body(xb_ref, top_ref, bot_ref, coords_ref, y_ref):
        i = pl.program_id(1)
        xt = jnp.concatenate([top_ref[0, :, 0], xb_ref[0], bot_ref[0, :, 0]],
                             axis=1)

        xl = jnp.concatenate([xt[:, :, :1], xt[:, :, :-1]], axis=2)
        xr = jnp.concatenate([xt[:, :, 1:], xt[:, :, -1:]], axis=2)

        wio = jax.lax.broadcasted_iota(jnp.int32, (1, 1, W), 2)
        xl_i = jnp.where(wio >= 1, xl, neg)
        xr_i = jnp.where(wio <= W - 2, xr, neg)
        rmax3 = jnp.maximum(jnp.maximum(xl_i, xr_i), xt)
        rmax2 = jnp.maximum(xl_i, xr_i)
        hio = i * TH + jax.lax.broadcasted_iota(jnp.int32, (1, TH, 1), 1)
        up8 = jnp.where(hio >= 1, rmax3[:, 0:TH], neg)
        dn8 = jnp.where(hio <= H - 2, rmax3[:, 2:TH + 2], neg)
        cen = xt[:, 1:TH + 1]
        m8 = jnp.maximum(jnp.maximum(up8, dn8), rmax2[:, 1:TH + 1])
        m9 = jnp.maximum(m8, cen)
        negp = jnp.full((1, TH, W), neg, dtype)
        m9u = jnp.concatenate([negp, m9[:-1]], axis=0)
        m9d = jnp.concatenate([m9[1:], negp], axis=0)
        m = jnp.maximum(jnp.maximum(m9u, m9d), m8)
        nms = cen > m

        xt_dm = jnp.concatenate([xt[:1], xt[:-1]], axis=0)
        xt_dp = jnp.concatenate([xt[1:], xt[-1:]], axis=0)
        xl_dm = jnp.concatenate([xl[:1], xl[:-1]], axis=0)
        xl_dp = jnp.concatenate([xl[1:], xl[-1:]], axis=0)
        xr_dm = jnp.concatenate([xr[:1], xr[:-1]], axis=0)
        xr_dp = jnp.concatenate([xr[1:], xr[-1:]], axis=0)

        def rows(a, o):
            return a[:, o:o + TH]

        dxg = 0.5 * (rows(xr, 1) - rows(xl, 1))
        dyg = 0.5 * (rows(xt, 2) - rows(xt, 0))
        dsg = 0.5 * (rows(xt_dp, 1) - rows(xt_dm, 1))
        dxx = rows(xr, 1) - 2.0 * cen + rows(xl, 1)
        dyy = rows(xt, 2) - 2.0 * cen + rows(xt, 0)
        dss = rows(xt_dp, 1) - 2.0 * cen + rows(xt_dm, 1)
        dxy = 0.25 * (rows(xr, 2) - rows(xl, 2) - rows(xr, 0) + rows(xl, 0))
        dys = 0.25 * (rows(xt_dp, 2) - rows(xt_dp, 0)
                      - rows(xt_dm, 2) + rows(xt_dm, 0))
        dxs = 0.25 * (rows(xr_dp, 1) - rows(xl_dp, 1)
                      - rows(xr_dm, 1) + rows(xl_dm, 1))

        a11 = dss + p[0][0]; a12 = dys + p[0][1]; a13 = dxs + p[0][2]
        a21 = dys + p[1][0]; a22 = dyy + p[1][1]; a23 = dxy + p[1][2]
        a31 = dxs + p[2][0]; a32 = dxy + p[2][1]; a33 = dss + p[2][2]
        cof11 = a22 * a33 - a23 * a32
        cof12 = a23 * a31 - a21 * a33
        cof13 = a21 * a32 - a22 * a31
        cof21 = a13 * a32 - a12 * a33
        cof22 = a11 * a33 - a13 * a31
        cof23 = a12 * a31 - a11 * a32
        cof31 = a12 * a23 - a13 * a22
        cof32 = a13 * a21 - a11 * a23
        cof33 = a11 * a22 - a12 * a21
        det = a11 * cof11 + a12 * cof12 + a13 * cof13
        rdet = 1.0 / det
        b1, b2, b3 = dsg, dyg, dxg
        s1 = (cof11 * b1 + cof21 * b2 + cof31 * b3) * rdet
        s2 = (cof12 * b1 + cof22 * b2 + cof32 * b3) * rdet
        s3 = (cof13 * b1 + cof23 * b2 + cof33 * b3) * rdet

        mab = jnp.maximum(jnp.maximum(jnp.abs(s1), jnp.abs(s2)), jnp.abs(s3))
        conv = jnp.logical_and(nms, mab < 0.5)
        d1 = jnp.where(conv, -s1, 0.0)
        d2 = jnp.where(conv, -s2, 0.0)
        d3 = jnp.where(conv, -s3, 0.0)
        dy = 0.5 * (b1 * d1 + b2 * d2 + b3 * d3)
        y_ref[0, 0] = cen + dy + STRICT_MAXIMA_BONUS * conv.astype(dtype)

        fdio = jax.lax.broadcasted_iota(jnp.int32, (D, TH, W), 0).astype(dtype)
        fwio = jax.lax.broadcasted_iota(jnp.int32, (D, TH, W), 2).astype(dtype)
        fhio = (i * TH
                + jax.lax.broadcasted_iota(jnp.int32, (D, TH, W), 1)
                ).astype(dtype)
        coords_ref[0, 0, 0] = fdio + d1
        coords_ref[0, 0, 1] = fhio + d2
        coords_ref[0, 0, 2] = fwio + d3

    return body


def kernel(x):
    B, C, D, H, W = x.shape
    dtype = x.dtype
    TH = 64
    nT = H // TH
    x4 = x.reshape(B, D, H, W)
    top = jnp.concatenate(
        [x4[:, :, :1], x4[:, :, TH - 1::TH][:, :, :nT - 1]], axis=2)
    bot = jnp.concatenate([x4[:, :, TH::TH], x4[:, :, H - 1:]], axis=2)
    top = top.reshape(B, D, nT, 1, W)
    bot = bot.reshape(B, D, nT, 1, W)

    body = _make_body(D, H, W, TH, dtype, _pert())
    coords, y = pl.pallas_call(
        body,
        grid=(B, nT),
        in_specs=[
            pl.BlockSpec((1, D, TH, W), lambda b, i: (b, 0, i, 0)),
            pl.BlockSpec((1, D, 1, 1, W), lambda b, i: (b, 0, i, 0, 0)),
            pl.BlockSpec((1, D, 1, 1, W), lambda b, i: (b, 0, i, 0, 0)),
        ],
        out_specs=[
            pl.BlockSpec((1, 1, 3, D, TH, W), lambda b, i: (b, 0, 0, 0, i, 0)),
            pl.BlockSpec((1, 1, D, TH, W), lambda b, i: (b, 0, 0, i, 0)),
        ],
        out_shape=[
            jax.ShapeDtypeStruct((B, 1, 3, D, H, W), dtype),
            jax.ShapeDtypeStruct((B, 1, D, H, W), dtype),
        ],
    )(x4, top, bot)
    return coords, y


# speedup vs baseline: 17039.1478x; 1.0082x over previous
import jax
import jax.numpy as jnp
from jax.experimental import pallas as pl
from jax.experimental.pallas import tpu as pltpu

STRICT_MAXIMA_BONUS = 10.0
EPS = 1e-07


def _make_body(D, H, W, TH, dtype):
    neg = float('-inf')

    def body(xb_ref, top_ref, bot_ref, pert_ref, coords_ref, y_ref):
        i = pl.program_id(1)
        p = [[pert_ref[r, c] for c in range(3)] for r in range(3)]
        xt = jnp.concatenate([top_ref[0, :, 0], xb_ref[0], bot_ref[0, :, 0]],
                             axis=1)

        xl = jnp.concatenate([xt[:, :, :1], xt[:, :, :-1]], axis=2)
        xr = jnp.concatenate([xt[:, :, 1:], xt[:, :, -1:]], axis=2)

        wio = jax.lax.broadcasted_iota(jnp.int32, (1, 1, W), 2)
        xl_i = jnp.where(wio >= 1, xl, neg)
        xr_i = jnp.where(wio <= W - 2, xr, neg)
        rmax3 = jnp.maximum(jnp.maximum(xl_i, xr_i), xt)
        rmax2 = jnp.maximum(xl_i, xr_i)
        hio = i * TH + jax.lax.broadcasted_iota(jnp.int32, (1, TH, 1), 1)
        up8 = jnp.where(hio >= 1, rmax3[:, 0:TH], neg)
        dn8 = jnp.where(hio <= H - 2, rmax3[:, 2:TH + 2], neg)
        cen = xt[:, 1:TH + 1]
        m8 = jnp.maximum(jnp.maximum(up8, dn8), rmax2[:, 1:TH + 1])
        m9 = jnp.maximum(m8, cen)
        negp = jnp.full((1, TH, W), neg, dtype)
        m9u = jnp.concatenate([negp, m9[:-1]], axis=0)
        m9d = jnp.concatenate([m9[1:], negp], axis=0)
        m = jnp.maximum(jnp.maximum(m9u, m9d), m8)
        nms = cen > m

        def rows(a, o):
            return a[:, o:o + TH]

        def dsh(a):
            return (jnp.concatenate([a[:1], a[:-1]], axis=0),
                    jnp.concatenate([a[1:], a[-1:]], axis=0))

        u1 = xr - xl
        sw = xr + xl
        xt_dm, xt_dp = dsh(xt)
        xd = xt_dp - xt_dm
        sd = xt_dp + xt_dm
        u1_dm, u1_dp = dsh(u1)
        ud = u1_dp - u1_dm

        dxg = 0.5 * rows(u1, 1)
        dyg = 0.5 * (rows(xt, 2) - rows(xt, 0))
        dsg = 0.5 * rows(xd, 1)
        dxx = rows(sw, 1) - 2.0 * cen
        dyy = rows(xt, 2) - 2.0 * cen + rows(xt, 0)
        dss = rows(sd, 1) - 2.0 * cen
        dxy = 0.25 * (rows(u1, 2) - rows(u1, 0))
        dys = 0.25 * (rows(xd, 2) - rows(xd, 0))
        dxs = 0.25 * rows(ud, 1)

        a11 = dss + p[0][0]; a12 = dys + p[0][1]; a13 = dxs + p[0][2]
        a21 = dys + p[1][0]; a22 = dyy + p[1][1]; a23 = dxy + p[1][2]
        a31 = dxs + p[2][0]; a32 = dxy + p[2][1]; a33 = dss + p[2][2]
        cof11 = a22 * a33 - a23 * a32
        cof12 = a23 * a31 - a21 * a33
        cof13 = a21 * a32 - a22 * a31
        cof21 = a13 * a32 - a12 * a33
        cof22 = a11 * a33 - a13 * a31
        cof23 = a12 * a31 - a11 * a32
        cof31 = a12 * a23 - a13 * a22
        cof32 = a13 * a21 - a11 * a23
        cof33 = a11 * a22 - a12 * a21
        det = a11 * cof11 + a12 * cof12 + a13 * cof13
        rdet = 1.0 / det
        b1, b2, b3 = dsg, dyg, dxg
        s1 = (cof11 * b1 + cof21 * b2 + cof31 * b3) * rdet
        s2 = (cof12 * b1 + cof22 * b2 + cof32 * b3) * rdet
        s3 = (cof13 * b1 + cof23 * b2 + cof33 * b3) * rdet

        mab = jnp.maximum(jnp.maximum(jnp.abs(s1), jnp.abs(s2)), jnp.abs(s3))
        conv = jnp.logical_and(nms, mab < 0.5)
        d1 = jnp.where(conv, -s1, 0.0)
        d2 = jnp.where(conv, -s2, 0.0)
        d3 = jnp.where(conv, -s3, 0.0)
        dy = 0.5 * (b1 * d1 + b2 * d2 + b3 * d3)
        y_ref[0, 0] = cen + dy + STRICT_MAXIMA_BONUS * conv.astype(dtype)

        fdio = jax.lax.broadcasted_iota(jnp.int32, (D, TH, W), 0).astype(dtype)
        fwio = jax.lax.broadcasted_iota(jnp.int32, (D, TH, W), 2).astype(dtype)
        fhio = (i * TH
                + jax.lax.broadcasted_iota(jnp.int32, (D, TH, W), 1)
                ).astype(dtype)
        coords_ref[0, 0, 0] = fdio + d1
        coords_ref[0, 0, 1] = fhio + d2
        coords_ref[0, 0, 2] = fwio + d3

    return body


def kernel(x):
    B, C, D, H, W = x.shape
    dtype = x.dtype
    TH = 128
    nT = H // TH
    x4 = x.reshape(B, D, H, W)
    top = jnp.concatenate(
        [x4[:, :, :1], x4[:, :, TH - 1::TH][:, :, :nT - 1]], axis=2)
    bot = jnp.concatenate([x4[:, :, TH::TH], x4[:, :, H - 1:]], axis=2)
    top = top.reshape(B, D, nT, 1, W)
    bot = bot.reshape(B, D, nT, 1, W)

    pert = jnp.abs(jax.random.uniform(
        jax.random.fold_in(jax.random.key(0), 7), (3, 3),
        dtype=dtype)) * EPS

    body = _make_body(D, H, W, TH, dtype)
    coords, y = pl.pallas_call(
        body,
        grid=(B, nT),
        in_specs=[
            pl.BlockSpec((1, D, TH, W), lambda b, i: (b, 0, i, 0)),
            pl.BlockSpec((1, D, 1, 1, W), lambda b, i: (b, 0, i, 0, 0)),
            pl.BlockSpec((1, D, 1, 1, W), lambda b, i: (b, 0, i, 0, 0)),
            pl.BlockSpec(memory_space=pltpu.SMEM),
        ],
        out_specs=[
            pl.BlockSpec((1, 1, 3, D, TH, W), lambda b, i: (b, 0, 0, 0, i, 0)),
            pl.BlockSpec((1, 1, D, TH, W), lambda b, i: (b, 0, 0, i, 0)),
        ],
        out_shape=[
            jax.ShapeDtypeStruct((B, 1, 3, D, H, W), dtype),
            jax.ShapeDtypeStruct((B, 1, D, H, W), dtype),
        ],
    )(x4, top, bot, pert)
    return coords, y


# speedup vs baseline: 20658.4641x; 1.2124x over previous
import jax
import jax.numpy as jnp
from jax.experimental import pallas as pl
from jax.experimental.pallas import tpu as pltpu

STRICT_MAXIMA_BONUS = 10.0
EPS = 1e-07


def _make_body(D, H, W, TH, dtype):
    neg = float('-inf')

    def body(xu_ref, xc_ref, xd_ref, pert_ref, coords_ref, y_ref):
        i = pl.program_id(1)
        p = [[pert_ref[r, c] for c in range(3)] for r in range(3)]
        xu = xu_ref[0]
        xc = xc_ref[0]
        xd = xd_ref[0]

        def lsh(a):
            return (jnp.concatenate([a[:, :, :1], a[:, :, :-1]], axis=2),
                    jnp.concatenate([a[:, :, 1:], a[:, :, -1:]], axis=2))

        def dsh(a):
            return (jnp.concatenate([a[:1], a[:-1]], axis=0),
                    jnp.concatenate([a[1:], a[-1:]], axis=0))

        xcl, xcr = lsh(xc)
        xul, xur = lsh(xu)
        xdl, xdr = lsh(xd)

        hio = i * TH + jax.lax.broadcasted_iota(jnp.int32, (1, TH, 1), 1)
        xu_m = jnp.where(hio >= 1, xu, neg)
        xd_m = jnp.where(hio <= H - 2, xd, neg)
        xul_m, xur_m = lsh(xu_m)
        xdl_m, xdr_m = lsh(xd_m)
        wio = jax.lax.broadcasted_iota(jnp.int32, (1, 1, W), 2)
        lmax = jnp.where(wio >= 1,
                         jnp.maximum(jnp.maximum(xul_m, xdl_m), xcl), neg)
        rmax = jnp.where(wio <= W - 2,
                         jnp.maximum(jnp.maximum(xur_m, xdr_m), xcr), neg)
        m8 = jnp.maximum(jnp.maximum(lmax, rmax), jnp.maximum(xu_m, xd_m))
        m9 = jnp.maximum(m8, xc)
        negp = jnp.full((1, TH, W), neg, dtype)
        m9u = jnp.concatenate([negp, m9[:-1]], axis=0)
        m9d = jnp.concatenate([m9[1:], negp], axis=0)
        m = jnp.maximum(jnp.maximum(m9u, m9d), m8)
        nms = xc > m

        u1 = xcr - xcl
        uh = xd - xu
        xc_dm, xc_dp = dsh(xc)
        uh_dm, uh_dp = dsh(uh)
        u1_dm, u1_dp = dsh(u1)

        dxg = 0.5 * u1
        dyg = 0.5 * uh
        dsg = 0.5 * (xc_dp - xc_dm)
        dxx = xcr + xcl - 2.0 * xc
        dyy = xd + xu - 2.0 * xc
        dss = xc_dp + xc_dm - 2.0 * xc
        dxy = 0.25 * ((xdr - xdl) - (xur - xul))
        dys = 0.25 * (uh_dp - uh_dm)
        dxs = 0.25 * (u1_dp - u1_dm)

        a11 = dss + p[0][0]; a12 = dys + p[0][1]; a13 = dxs + p[0][2]
        a21 = dys + p[1][0]; a22 = dyy + p[1][1]; a23 = dxy + p[1][2]
        a31 = dxs + p[2][0]; a32 = dxy + p[2][1]; a33 = dss + p[2][2]
        cof11 = a22 * a33 - a23 * a32
        cof12 = a23 * a31 - a21 * a33
        cof13 = a21 * a32 - a22 * a31
        cof21 = a13 * a32 - a12 * a33
        cof22 = a11 * a33 - a13 * a31
        cof23 = a12 * a31 - a11 * a32
        cof31 = a12 * a23 - a13 * a22
        cof32 = a13 * a21 - a11 * a23
        cof33 = a11 * a22 - a12 * a21
        det = a11 * cof11 + a12 * cof12 + a13 * cof13
        rdet = 1.0 / det
        b1, b2, b3 = dsg, dyg, dxg
        s1 = (cof11 * b1 + cof21 * b2 + cof31 * b3) * rdet
        s2 = (cof12 * b1 + cof22 * b2 + cof32 * b3) * rdet
        s3 = (cof13 * b1 + cof23 * b2 + cof33 * b3) * rdet

        mab = jnp.maximum(jnp.maximum(jnp.abs(s1), jnp.abs(s2)), jnp.abs(s3))
        conv = jnp.logical_and(nms, mab < 0.5)
        d1 = jnp.where(conv, -s1, 0.0)
        d2 = jnp.where(conv, -s2, 0.0)
        d3 = jnp.where(conv, -s3, 0.0)
        dy = 0.5 * (b1 * d1 + b2 * d2 + b3 * d3)
        y_ref[0, 0] = xc + dy + STRICT_MAXIMA_BONUS * conv.astype(dtype)

        fdio = jax.lax.broadcasted_iota(jnp.int32, (D, TH, W), 0).astype(dtype)
        fwio = jax.lax.broadcasted_iota(jnp.int32, (D, TH, W), 2).astype(dtype)
        fhio = (i * TH
                + jax.lax.broadcasted_iota(jnp.int32, (D, TH, W), 1)
                ).astype(dtype)
        coords_ref[0, 0, 0] = fdio + d1
        coords_ref[0, 0, 1] = fhio + d2
        coords_ref[0, 0, 2] = fwio + d3

    return body


def kernel(x):
    B, C, D, H, W = x.shape
    dtype = x.dtype
    TH = 128
    nT = H // TH
    x4 = x.reshape(B, D, H, W)
    xu = jnp.concatenate([x4[:, :, :1], x4[:, :, :H - 1]], axis=2)
    xd = jnp.concatenate([x4[:, :, 1:], x4[:, :, H - 1:]], axis=2)

    pert = jnp.abs(jax.random.uniform(
        jax.random.fold_in(jax.random.key(0), 7), (3, 3),
        dtype=dtype)) * EPS

    spec = pl.BlockSpec((1, D, TH, W), lambda b, i: (b, 0, i, 0))
    body = _make_body(D, H, W, TH, dtype)
    coords, y = pl.pallas_call(
        body,
        grid=(B, nT),
        in_specs=[
            spec,
            spec,
            spec,
            pl.BlockSpec(memory_space=pltpu.SMEM),
        ],
        out_specs=[
            pl.BlockSpec((1, 1, 3, D, TH, W), lambda b, i: (b, 0, 0, 0, i, 0)),
            pl.BlockSpec((1, 1, D, TH, W), lambda b, i: (b, 0, 0, i, 0)),
        ],
        out_shape=[
            jax.ShapeDtypeStruct((B, 1, 3, D, H, W), dtype),
            jax.ShapeDtypeStruct((B, 1, D, H, W), dtype),
        ],
    )(xu, x4, xd, pert)
    return coords, y
